# baseline (speedup 1.0000x reference)
import functools

import jax
import jax.numpy as jnp
from jax import lax
from jax.experimental import pallas as pl
from jax.experimental.pallas import tpu as pltpu
from jax.experimental.pallas import tpu_sc as plsc

_LANES = 16


def _sc_quantize(total, n_workers):
    chunk = total // n_workers
    n_vecs = chunk // _LANES
    mesh = plsc.VectorSubcoreMesh(core_axis_name="c", subcore_axis_name="s")

    @functools.partial(
        pl.kernel,
        out_type=[
            jax.ShapeDtypeStruct((n_workers, chunk), jnp.float32),
            jax.ShapeDtypeStruct((n_workers, chunk), jnp.float32),
            jax.ShapeDtypeStruct((n_workers, chunk), jnp.int32),
        ],
        mesh=mesh,
        compiler_params=pltpu.CompilerParams(needs_layout_passes=False),
        scratch_types=[
            pltpu.VMEM((chunk,), jnp.float32),
            pltpu.VMEM((64,), jnp.float32),
            pltpu.VMEM((64,), jnp.float32),
            pltpu.VMEM((chunk,), jnp.float32),
            pltpu.VMEM((chunk,), jnp.int32),
        ],
    )
    def body(x_hbm, mids_hbm, cent_hbm, ste_hbm, hard_hbm, idx_hbm,
             x_v, mids_v, cent_v, hard_v, idx_v):
        wid = lax.axis_index("s") * 2 + lax.axis_index("c")
        pltpu.sync_copy(mids_hbm, mids_v)
        pltpu.sync_copy(cent_hbm, cent_v)
        pltpu.sync_copy(x_hbm.at[wid], x_v)

        @plsc.parallel_loop(0, n_vecs, 1, unroll=8)
        def step(i):
            off = i * _LANES
            xv = x_v[pl.ds(off, _LANES)]
            pos = jnp.zeros((_LANES,), jnp.int32)
            for s in (32, 16, 8, 4, 2, 1):
                t = plsc.load_gather(mids_v, [pos + (s - 1)])
                pos = pos + jnp.where(xv > t, jnp.int32(s), jnp.int32(0))
            hard = plsc.load_gather(cent_v, [pos])
            hard_v[pl.ds(off, _LANES)] = hard
            idx_v[pl.ds(off, _LANES)] = pos

        pltpu.sync_copy(hard_v, ste_hbm.at[wid])
        pltpu.sync_copy(hard_v, hard_hbm.at[wid])
        pltpu.sync_copy(idx_v, idx_hbm.at[wid])

    return body


def kernel(x, centers):
    n, c, h, w = x.shape
    total = n * c * h * w
    n_workers = 32
    xf = x.reshape(n_workers, total // n_workers)
    mids = jnp.concatenate(
        [(centers[:-1] + centers[1:]) * 0.5, jnp.full((1,), jnp.inf, jnp.float32)]
    )
    ste, hard, idx = _sc_quantize(total, n_workers)(xf, mids, centers)
    shape = (n, c, h, w)
    return (ste.reshape(shape), hard.reshape(shape), idx.reshape(shape))


# speedup vs baseline: 1.4544x; 1.4544x over previous
import functools

import jax
import jax.numpy as jnp
from jax import lax
from jax.experimental import pallas as pl
from jax.experimental.pallas import tpu as pltpu
from jax.experimental.pallas import tpu_sc as plsc

_LANES = 16


def _sc_quantize(total, n_workers):
    chunk = total // n_workers
    n_vecs = chunk // _LANES
    mesh = plsc.VectorSubcoreMesh(core_axis_name="c", subcore_axis_name="s")

    @functools.partial(
        pl.kernel,
        out_type=[
            jax.ShapeDtypeStruct((total,), jnp.float32),
            jax.ShapeDtypeStruct((total,), jnp.int32),
        ],
        mesh=mesh,
        compiler_params=pltpu.CompilerParams(needs_layout_passes=False),
        scratch_types=[
            pltpu.VMEM((chunk,), jnp.float32),
            pltpu.VMEM((64,), jnp.float32),
            pltpu.VMEM((64,), jnp.float32),
            pltpu.VMEM((chunk,), jnp.float32),
            pltpu.VMEM((chunk,), jnp.int32),
        ],
    )
    def body(x_hbm, mids_hbm, cent_hbm, hard_hbm, idx_hbm,
             x_v, mids_v, cent_v, hard_v, idx_v):
        wid = lax.axis_index("s") * 2 + lax.axis_index("c")
        base = wid * chunk
        pltpu.sync_copy(mids_hbm, mids_v)
        pltpu.sync_copy(cent_hbm, cent_v)
        pltpu.sync_copy(x_hbm.at[pl.ds(base, chunk)], x_v)

        @plsc.parallel_loop(0, n_vecs, 1, unroll=8)
        def step(i):
            off = i * _LANES
            xv = x_v[pl.ds(off, _LANES)]
            pos = jnp.zeros((_LANES,), jnp.int32)
            for s in (32, 16, 8, 4, 2, 1):
                t = plsc.load_gather(mids_v, [pos + (s - 1)])
                pos = pos + jnp.where(xv > t, jnp.int32(s), jnp.int32(0))
            hard = plsc.load_gather(cent_v, [pos])
            hard_v[pl.ds(off, _LANES)] = hard
            idx_v[pl.ds(off, _LANES)] = pos

        pltpu.sync_copy(hard_v, hard_hbm.at[pl.ds(base, chunk)])
        pltpu.sync_copy(idx_v, idx_hbm.at[pl.ds(base, chunk)])

    return body


def kernel(x, centers):
    n, c, h, w = x.shape
    total = n * c * h * w
    xf = x.reshape(total)
    mids = jnp.concatenate(
        [(centers[:-1] + centers[1:]) * 0.5, jnp.full((1,), jnp.inf, jnp.float32)]
    )
    hard, idx = _sc_quantize(total, 32)(xf, mids, centers)
    shape = (n, c, h, w)
    hard4 = hard.reshape(shape)
    return (hard4, hard4, idx.reshape(shape))


# speedup vs baseline: 1.7932x; 1.2329x over previous
import functools

import jax
import jax.numpy as jnp
from jax import lax
from jax.experimental import pallas as pl
from jax.experimental.pallas import tpu as pltpu
from jax.experimental.pallas import tpu_sc as plsc

_LANES = 16
_HW = 24
_SLAB = _HW * _HW
_VPS = _SLAB // _LANES


def _sc_quantize(n_slabs, n_workers, slabs_per_round):
    slabs_per_worker = n_slabs // n_workers
    n_rounds = slabs_per_worker // slabs_per_round
    vecs_per_round = slabs_per_round * _VPS
    mesh = plsc.VectorSubcoreMesh(core_axis_name="c", subcore_axis_name="s")

    @functools.partial(
        pl.kernel,
        out_type=[
            jax.ShapeDtypeStruct((n_slabs, _HW, _HW), jnp.float32),
            jax.ShapeDtypeStruct((n_slabs, _HW, _HW), jnp.int32),
        ],
        mesh=mesh,
        compiler_params=pltpu.CompilerParams(
            needs_layout_passes=False, use_tc_tiling_on_sc=True
        ),
        scratch_types=[
            pltpu.VMEM((slabs_per_round, _HW, _HW), jnp.float32),
            pltpu.VMEM((slabs_per_round, _HW, _HW), jnp.float32),
            pltpu.VMEM((slabs_per_round, _HW, _HW), jnp.int32),
            pltpu.VMEM((_SLAB,), jnp.int32),
            pltpu.VMEM((_SLAB,), jnp.int32),
            pltpu.VMEM((64,), jnp.float32),
            pltpu.VMEM((64,), jnp.float32),
        ],
    )
    def body(x_hbm, ri_hbm, ci_hbm, mids_hbm, cent_hbm, hard_hbm, idx_hbm,
             x_v, hard_v, idx_v, ri_v, ci_v, mids_v, cent_v):
        wid = lax.axis_index("s") * 2 + lax.axis_index("c")
        base = wid * slabs_per_worker
        pltpu.sync_copy(mids_hbm, mids_v)
        pltpu.sync_copy(cent_hbm, cent_v)
        pltpu.sync_copy(ri_hbm, ri_v)
        pltpu.sync_copy(ci_hbm, ci_v)

        def do_round(r, _):
            s0 = base + r * slabs_per_round
            pltpu.sync_copy(x_hbm.at[pl.ds(s0, slabs_per_round)], x_v)

            @plsc.parallel_loop(0, vecs_per_round, 1, unroll=4)
            def step(i):
                s = i // _VPS
                t = (i % _VPS) * _LANES
                sv = jnp.full((_LANES,), s, jnp.int32)
                ri = ri_v[pl.ds(t, _LANES)]
                ci = ci_v[pl.ds(t, _LANES)]
                xv = plsc.load_gather(x_v, [sv, ri, ci])
                pos = jnp.zeros((_LANES,), jnp.int32)
                for st in (32, 16, 8, 4, 2, 1):
                    m = plsc.load_gather(mids_v, [pos + (st - 1)])
                    pos = pos + jnp.where(xv > m, jnp.int32(st), jnp.int32(0))
                hard = plsc.load_gather(cent_v, [pos])
                plsc.store_scatter(hard_v, [sv, ri, ci], hard)
                plsc.store_scatter(idx_v, [sv, ri, ci], pos)

            pltpu.sync_copy(hard_v, hard_hbm.at[pl.ds(s0, slabs_per_round)])
            pltpu.sync_copy(idx_v, idx_hbm.at[pl.ds(s0, slabs_per_round)])
            return _

        lax.fori_loop(0, n_rounds, do_round, None)

    return body


def kernel(x, centers):
    n, c, h, w = x.shape
    n_slabs = n * c
    x3 = x.reshape(n_slabs, h, w)
    mids = jnp.concatenate(
        [(centers[:-1] + centers[1:]) * 0.5, jnp.full((1,), jnp.inf, jnp.float32)]
    )
    q = jnp.arange(_SLAB, dtype=jnp.int32)
    ri = q // _HW
    ci = q % _HW
    hard, idx = _sc_quantize(n_slabs, 32, 8)(x3, ri, ci, mids, centers)
    shape = (n, c, h, w)
    hard4 = hard.reshape(shape)
    return (hard4, hard4, idx.reshape(shape))


# speedup vs baseline: 2.0135x; 1.1229x over previous
import functools

import jax
import jax.numpy as jnp
from jax import lax
from jax.experimental import pallas as pl
from jax.experimental.pallas import tpu as pltpu
from jax.experimental.pallas import tpu_sc as plsc

_LANES = 16
_HW = 24


def _sc_quantize(n, c):
    n_workers = 32
    slabs_per_worker = (n * c) // n_workers
    spr = 6
    n_rounds = slabs_per_worker // spr
    rows_per_round = spr * _HW
    workers_per_n = c // slabs_per_worker
    mesh = plsc.VectorSubcoreMesh(core_axis_name="c", subcore_axis_name="s")

    out_f32 = jax.ShapeDtypeStruct((n, c, _HW, _HW), jnp.float32)
    out_i32 = jax.ShapeDtypeStruct((n, c, _HW, _HW), jnp.int32)

    @functools.partial(
        pl.kernel,
        out_type=[out_f32, out_f32, out_i32],
        mesh=mesh,
        compiler_params=pltpu.CompilerParams(
            needs_layout_passes=False, use_tc_tiling_on_sc=True
        ),
        scratch_types=[
            pltpu.VMEM((spr, _HW, _HW), jnp.float32),
            pltpu.VMEM((spr, _HW, _HW), jnp.float32),
            pltpu.VMEM((spr, _HW, _HW), jnp.float32),
            pltpu.VMEM((spr, _HW, _HW), jnp.float32),
            pltpu.VMEM((spr, _HW, _HW), jnp.int32),
            pltpu.VMEM((spr, _HW, _HW), jnp.int32),
            pltpu.VMEM((64,), jnp.float32),
            pltpu.VMEM((64,), jnp.float32),
            pltpu.SemaphoreType.DMA,
            pltpu.SemaphoreType.DMA,
            pltpu.SemaphoreType.DMA,
            pltpu.SemaphoreType.DMA,
        ],
    )
    def body(x_hbm, mids_hbm, cent_hbm, ste_hbm, hard_hbm, idx_hbm,
             x0, x1, h0, h1, i0, i1, mids_v, cent_v,
             sem_in0, sem_in1, sem_out0, sem_out1):
        x_b, h_b, i_b = (x0, x1), (h0, h1), (i0, i1)
        sem_in, sem_out = (sem_in0, sem_in1), (sem_out0, sem_out1)
        wid = lax.axis_index("s") * 2 + lax.axis_index("c")
        n0 = wid // workers_per_n
        c_base = (wid % workers_per_n) * slabs_per_worker
        pltpu.sync_copy(mids_hbm, mids_v)
        pltpu.sync_copy(cent_hbm, cent_v)

        def splat(k):
            return plsc.load_gather(mids_v, [jnp.full((_LANES,), k, jnp.int32)])

        m31 = splat(31)
        m15, m47 = splat(15), splat(47)
        m7, m23, m39, m55 = splat(7), splat(23), splat(39), splat(55)

        def search(xv):
            b32 = xv > m31
            t16 = jnp.where(b32, m47, m15)
            b16 = xv > t16
            ta = jnp.where(b32, m39, m7)
            tb = jnp.where(b32, m55, m23)
            b8 = xv > jnp.where(b16, tb, ta)
            pos = (jnp.where(b32, jnp.int32(32), jnp.int32(0))
                   + jnp.where(b16, jnp.int32(16), jnp.int32(0))
                   + jnp.where(b8, jnp.int32(8), jnp.int32(0)))
            for st in (4, 2, 1):
                m = plsc.load_gather(mids_v, [pos + (st - 1)])
                pos = pos + jnp.where(xv > m, jnp.int32(st), jnp.int32(0))
            return pos, plsc.load_gather(cent_v, [pos])

        def slab_slice(r):
            return pl.ds(c_base + r * spr, spr)

        h_in = [None, None]
        h_out = [None] * n_rounds
        h_in[0] = pltpu.async_copy(x_hbm.at[n0, slab_slice(0)], x_b[0], sem_in[0])
        for r in range(n_rounds):
            b = r % 2
            if r + 1 < n_rounds:
                h_in[1 - b] = pltpu.async_copy(
                    x_hbm.at[n0, slab_slice(r + 1)], x_b[1 - b], sem_in[1 - b])
            h_in[b].wait()
            if r >= 2:
                for h in h_out[r - 2]:
                    h.wait()
            x_v, hard_v, idx_v = x_b[b], h_b[b], i_b[b]

            @plsc.parallel_loop(0, rows_per_round, 1, unroll=4)
            def row(i):
                s = i // _HW
                rr = i % _HW
                for off in (0, 8):
                    xv = x_v[s, rr, pl.ds(off, _LANES)]
                    pos, hard = search(xv)
                    hard_v[s, rr, pl.ds(off, _LANES)] = hard
                    idx_v[s, rr, pl.ds(off, _LANES)] = pos

            sl = slab_slice(r)
            h_out[r] = [
                pltpu.async_copy(hard_v, ste_hbm.at[n0, sl], sem_out[b]),
                pltpu.async_copy(hard_v, hard_hbm.at[n0, sl], sem_out[b]),
                pltpu.async_copy(idx_v, idx_hbm.at[n0, sl], sem_out[b]),
            ]
        for r in (n_rounds - 2, n_rounds - 1):
            for h in h_out[r]:
                h.wait()

    return body


def kernel(x, centers):
    n, c, h, w = x.shape
    mids = jnp.concatenate(
        [(centers[:-1] + centers[1:]) * 0.5, jnp.full((1,), jnp.inf, jnp.float32)]
    )
    ste, hard, idx = _sc_quantize(n, c)(x, mids, centers)
    return (ste, hard, idx)
